# baseline (speedup 1.0000x reference)
import math

import jax
import jax.numpy as jnp
from jax import lax
from jax.experimental import pallas as pl
from jax.experimental.pallas import tpu as pltpu
from jax.experimental.pallas import tpu_sc as plsc

T = 2048
HIDDEN = 768
NH = 12
NKV = 6
HD = 64
E = 8
TOPK = 2
IM = 2048
EPS = 1e-05
BASE = 10000.0

QW = NH * HD
KW = NKV * HD
ROPE_W = QW + KW
QKV_W = QW + 2 * KW
HALF = HD // 2

BLK = 256
NBMAX = 24
PADT = NBMAX * BLK

NW = 32
CHUNK = T // NW

_NEG = -1e30
_BF = jnp.bfloat16


def _rms(x, w):
    v = jnp.mean(jnp.square(x), axis=-1, keepdims=True)
    return x * jax.lax.rsqrt(v + EPS) * w


def _bdot_t(a, b):
    return jax.lax.dot_general(a.astype(_BF), b.astype(_BF),
                               (((1,), (1,)), ((), ())),
                               preferred_element_type=jnp.float32)


def _bdot(a, b):
    return jnp.dot(a.astype(_BF), b.astype(_BF),
                   preferred_element_type=jnp.float32)


def _trig_kernel(posf_ref, trig_ref):
    pos = posf_ref[...]
    ci = (jax.lax.broadcasted_iota(jnp.int32, (T, 128), 1)
          % HALF).astype(jnp.float32)
    inv = jnp.exp(ci * (-math.log(BASE) / HALF))
    fr = pos * inv
    trig_ref[:, :128] = jnp.cos(fr)
    trig_ref[:, 128:] = jnp.sin(fr)


def _rope_table(posf):
    return pl.pallas_call(
        _trig_kernel,
        grid=(1,),
        in_specs=[pl.BlockSpec((T, 1), lambda i: (0, 0))],
        out_specs=pl.BlockSpec((T, 256), lambda i: (0, 0)),
        out_shape=jax.ShapeDtypeStruct((T, 256), jnp.float32),
    )(posf)


def _qkv_kernel(trig_ref, x_ref, wpre_ref, wqkv_ref, qkv_ref):
    x = x_ref[...]
    h = _rms(x, wpre_ref[...])
    qkv = _bdot_t(h, wqkv_ref[...])

    bt = qkv.shape[0]
    cosf = jnp.concatenate([trig_ref[:, :128]] * (ROPE_W // 128), axis=1)
    sinf = jnp.concatenate([trig_ref[:, 128:]] * (ROPE_W // 128), axis=1)
    col = jax.lax.broadcasted_iota(jnp.int32, (bt, ROPE_W), 1)
    first = (col % HD) < HALF

    reg = qkv[:, :ROPE_W]
    plus = qkv[:, HALF:ROPE_W + HALF]
    minus = jnp.concatenate([qkv[:, :HALF], qkv[:, :ROPE_W - HALF]], axis=1)
    rot = jnp.where(first, -plus, minus) * sinf
    roped = reg * cosf + rot
    qkv_ref[...] = jnp.concatenate([roped, qkv[:, ROPE_W:]], axis=1)


def _qkv_rope(trig, x, w_pre_attn, wqkv):
    bt = 256
    return pl.pallas_call(
        _qkv_kernel,
        grid=(T // bt,),
        in_specs=[
            pl.BlockSpec((bt, 256), lambda t: (t, 0)),
            pl.BlockSpec((bt, HIDDEN), lambda t: (t, 0)),
            pl.BlockSpec((1, HIDDEN), lambda t: (0, 0)),
            pl.BlockSpec((QKV_W, HIDDEN), lambda t: (0, 0)),
        ],
        out_specs=pl.BlockSpec((bt, QKV_W), lambda t: (t, 0)),
        out_shape=jax.ShapeDtypeStruct((T, QKV_W), jnp.float32),
    )(trig, x, w_pre_attn, wqkv)


BQ = 1024


def _attn_block(qblk, kfull, vfull, qbase, klen):
    outs = []
    row = jax.lax.broadcasted_iota(jnp.int32, (BQ, klen), 0) + qbase
    colc = jax.lax.broadcasted_iota(jnp.int32, (BQ, klen), 1)
    causal = row >= colc
    scale = HD ** -0.5
    for hh in range(4):
        q = qblk[:, hh * HD:(hh + 1) * HD]
        kvo = (hh // 2) * HD
        k = kfull[:klen, kvo:kvo + HD]
        v = vfull[:klen, kvo:kvo + HD]
        s = _bdot_t(q, k) * scale
        s = jnp.where(causal, s, _NEG)
        m = jnp.max(s, axis=-1, keepdims=True)
        p = jnp.exp(s - m)
        p = p / jnp.sum(p, axis=-1, keepdims=True)
        outs.append(_bdot(p, v))
    return jnp.concatenate(outs, axis=1)


def _attn_kernel(q_ref, k_ref, v_ref, o_ref):
    qi = pl.program_id(1)

    @pl.when(qi == 0)
    def _():
        o_ref[...] = _attn_block(q_ref[...], k_ref[...], v_ref[...], 0, BQ)

    @pl.when(qi == 1)
    def _():
        o_ref[...] = _attn_block(q_ref[...], k_ref[...], v_ref[...], BQ, T)


def _attention(qkv):
    return pl.pallas_call(
        _attn_kernel,
        grid=(NH // 4, T // BQ),
        in_specs=[
            pl.BlockSpec((BQ, 4 * HD), lambda g, q: (q, g)),
            pl.BlockSpec((T, 2 * HD), lambda g, q: (0, QW // 128 + g)),
            pl.BlockSpec((T, 2 * HD), lambda g, q: (0, (QW + KW) // 128 + g)),
        ],
        out_specs=pl.BlockSpec((BQ, 4 * HD), lambda g, q: (q, g)),
        out_shape=jax.ShapeDtypeStruct((T, QW), jnp.float32),
    )(qkv, qkv, qkv)


def _post_kernel(o_ref, hs_ref, wo_ref, wpost_ref, wpremoe_ref, gw_ref,
                 resid_ref, xm_ref, logt_ref):
    a = _bdot_t(o_ref[...], wo_ref[...])
    added = a + hs_ref[...]
    h = _rms(added, wpost_ref[...])
    resid_ref[...] = h
    xm = _rms(h, wpremoe_ref[...])
    xm_ref[...] = xm
    logt_ref[...] = jax.lax.dot_general(
        gw_ref[...].astype(_BF), xm.astype(_BF), (((1,), (1,)), ((), ())),
        preferred_element_type=jnp.float32)


def _post_attn(o, hs, wo, w_post_attn, w_pre_moe, gate_w):
    bt = 512
    return pl.pallas_call(
        _post_kernel,
        grid=(T // bt,),
        in_specs=[
            pl.BlockSpec((bt, QW), lambda t: (t, 0)),
            pl.BlockSpec((bt, HIDDEN), lambda t: (t, 0)),
            pl.BlockSpec((HIDDEN, QW), lambda t: (0, 0)),
            pl.BlockSpec((1, HIDDEN), lambda t: (0, 0)),
            pl.BlockSpec((1, HIDDEN), lambda t: (0, 0)),
            pl.BlockSpec((E, HIDDEN), lambda t: (0, 0)),
        ],
        out_specs=[
            pl.BlockSpec((bt, HIDDEN), lambda t: (t, 0)),
            pl.BlockSpec((bt, HIDDEN), lambda t: (t, 0)),
            pl.BlockSpec((E, bt), lambda t: (0, t)),
        ],
        out_shape=[
            jax.ShapeDtypeStruct((T, HIDDEN), jnp.float32),
            jax.ShapeDtypeStruct((T, HIDDEN), jnp.float32),
            jax.ShapeDtypeStruct((E, T), jnp.float32),
        ],
    )(o, hs, wo, w_post_attn, w_pre_moe, gate_w)


def _dispatch_kernel(logt_ref, dispi_ref, dispw_ref):
    lt = logt_ref[...]
    mx = jnp.max(lt, axis=0, keepdims=True)
    p = jnp.exp(lt - mx)
    probs = p / jnp.sum(p, axis=0, keepdims=True)

    e_col = jax.lax.broadcasted_iota(jnp.int32, (E, T), 0)
    m1 = jnp.max(probs, axis=0, keepdims=True)
    i1 = jnp.min(jnp.where(probs == m1, e_col, E), axis=0, keepdims=True)
    masked = jnp.where(e_col == i1, -1.0, probs)
    m2 = jnp.max(masked, axis=0, keepdims=True)
    i2 = jnp.min(jnp.where(masked == m2, e_col, E), axis=0, keepdims=True)
    wsum = m1 + m2
    w1 = m1 / wsum
    w2 = m2 / wsum

    ind = jnp.where((e_col == i1) | (e_col == i2), 1.0, 0.0)

    r_iota = jax.lax.broadcasted_iota(jnp.int32, (T, T), 0)
    c_iota = jax.lax.broadcasted_iota(jnp.int32, (T, T), 1)
    tri = jnp.where(r_iota <= c_iota, 1.0, 0.0)
    cum = jax.lax.dot_general(ind.astype(_BF), tri.astype(_BF),
                              (((1,), (0,)), ((), ())),
                              preferred_element_type=jnp.float32)
    rank = cum - ind
    counts = cum[:, T - 1:T]
    nblk = jnp.floor((counts + (BLK - 1)) * (1.0 / BLK))
    l8r = jax.lax.broadcasted_iota(jnp.int32, (E, E), 0)
    l8c = jax.lax.broadcasted_iota(jnp.int32, (E, E), 1)
    lower8 = jnp.where(l8r >= l8c, 1.0, 0.0)
    blkinc = jax.lax.dot_general(lower8.astype(_BF), nblk.astype(_BF),
                                 (((1,), (0,)), ((), ())),
                                 preferred_element_type=jnp.float32)
    padoff = (blkinc - nblk) * float(BLK)

    dall = padoff + rank
    dest1 = jnp.sum(jnp.where(e_col == i1, dall, 0.0), axis=0, keepdims=True)
    dest2 = jnp.sum(jnp.where(e_col == i2, dall, 0.0), axis=0, keepdims=True)

    nbtot = blkinc[E - 1:E, :]
    colt = jax.lax.broadcasted_iota(jnp.int32, (1, T), 1).astype(jnp.float32)
    bc = jnp.minimum(colt, nbtot - 1.0)
    be = jnp.sum(jnp.where(blkinc <= bc, 1.0, 0.0), axis=0, keepdims=True)

    rowi = jax.lax.broadcasted_iota(jnp.int32, (4, T), 0)
    out = jnp.where(rowi == 0, dest1,
          jnp.where(rowi == 1, dest2,
          jnp.where(rowi == 2, be, nbtot)))
    dispi_ref[...] = out.astype(jnp.int32)

    rowi2 = jax.lax.broadcasted_iota(jnp.int32, (2, T), 0)
    dispw_ref[...] = jnp.where(rowi2 == 0, w1, w2)


def _dispatch_indices(logt):
    return pl.pallas_call(
        _dispatch_kernel,
        grid=(1,),
        in_specs=[pl.BlockSpec((E, T), lambda i: (0, 0))],
        out_specs=[
            pl.BlockSpec((4, T), lambda i: (0, 0)),
            pl.BlockSpec((2, T), lambda i: (0, 0)),
        ],
        out_shape=[
            jax.ShapeDtypeStruct((4, T), jnp.int32),
            jax.ShapeDtypeStruct((2, T), jnp.float32),
        ],
    )(logt)


def _sc_wid():
    return lax.axis_index("s") * 2 + lax.axis_index("c")


def _sc_dispatch(xm, dispi, dispw):
    def body(xm_hbm, dispi_hbm, dispw_hbm, xs_hbm, wslot_hbm,
             d1_v, d2_v, w1_v, w2_v, rows_v, sem1, sem2, sem3, sem4):
        base = _sc_wid() * CHUNK
        pltpu.sync_copy(dispi_hbm.at[0, pl.ds(base, CHUNK)], d1_v)
        pltpu.sync_copy(dispi_hbm.at[1, pl.ds(base, CHUNK)], d2_v)
        pltpu.sync_copy(dispw_hbm.at[0, pl.ds(base, CHUNK)], w1_v)
        pltpu.sync_copy(dispw_hbm.at[1, pl.ds(base, CHUNK)], w2_v)
        pltpu.sync_copy(xm_hbm.at[pl.ds(base, CHUNK)], rows_v)
        c1 = pltpu.async_copy(rows_v, xs_hbm.at[d1_v], sem1)
        c2 = pltpu.async_copy(rows_v, xs_hbm.at[d2_v], sem2)
        c3 = pltpu.async_copy(w1_v, wslot_hbm.at[d1_v], sem3)
        c4 = pltpu.async_copy(w2_v, wslot_hbm.at[d2_v], sem4)
        c1.wait()
        c2.wait()
        c3.wait()
        c4.wait()

    return pl.kernel(
        body,
        out_type=(jax.ShapeDtypeStruct((PADT, HIDDEN), jnp.float32),
                  jax.ShapeDtypeStruct((PADT,), jnp.float32)),
        mesh=plsc.VectorSubcoreMesh(core_axis_name="c", subcore_axis_name="s"),
        scratch_types=[
            pltpu.VMEM((CHUNK,), jnp.int32),
            pltpu.VMEM((CHUNK,), jnp.int32),
            pltpu.VMEM((CHUNK,), jnp.float32),
            pltpu.VMEM((CHUNK,), jnp.float32),
            pltpu.VMEM((CHUNK, HIDDEN), jnp.float32),
            pltpu.SemaphoreType.DMA,
            pltpu.SemaphoreType.DMA,
            pltpu.SemaphoreType.DMA,
            pltpu.SemaphoreType.DMA,
        ],
    )(xm, dispi, dispw)


def _sc_combine(ys, dispi):
    def body(ys_hbm, dispi_hbm, y1_hbm, y2_hbm,
             d1_v, d2_v, g1_v, g2_v, sem1, sem2):
        base = _sc_wid() * CHUNK
        pltpu.sync_copy(dispi_hbm.at[0, pl.ds(base, CHUNK)], d1_v)
        pltpu.sync_copy(dispi_hbm.at[1, pl.ds(base, CHUNK)], d2_v)
        c1 = pltpu.async_copy(ys_hbm.at[d1_v], g1_v, sem1)
        c2 = pltpu.async_copy(ys_hbm.at[d2_v], g2_v, sem2)
        c1.wait()
        c2.wait()
        pltpu.sync_copy(g1_v, y1_hbm.at[pl.ds(base, CHUNK)])
        pltpu.sync_copy(g2_v, y2_hbm.at[pl.ds(base, CHUNK)])

    return pl.kernel(
        body,
        out_type=(jax.ShapeDtypeStruct((T, HIDDEN), jnp.float32),
                  jax.ShapeDtypeStruct((T, HIDDEN), jnp.float32)),
        mesh=plsc.VectorSubcoreMesh(core_axis_name="c", subcore_axis_name="s"),
        scratch_types=[
            pltpu.VMEM((CHUNK,), jnp.int32),
            pltpu.VMEM((CHUNK,), jnp.int32),
            pltpu.VMEM((CHUNK, HIDDEN), jnp.float32),
            pltpu.VMEM((CHUNK, HIDDEN), jnp.float32),
            pltpu.SemaphoreType.DMA,
            pltpu.SemaphoreType.DMA,
        ],
    )(ys, dispi)


def _moe_kernel(be_ref, nb_ref, xs_ref, wsl_ref, wsg_ref, wsu_ref, w2s_ref,
                ys_ref):
    b = pl.program_id(0)

    @pl.when(b < nb_ref[0])
    def _():
        xb = xs_ref[...].astype(_BF)
        g = jax.lax.dot_general(xb, wsg_ref[0].astype(_BF),
                                (((1,), (1,)), ((), ())),
                                preferred_element_type=jnp.float32)
        u = jax.lax.dot_general(xb, wsu_ref[0].astype(_BF),
                                (((1,), (1,)), ((), ())),
                                preferred_element_type=jnp.float32)
        sig = 1.0 / (1.0 + jnp.exp(-g))
        h = (g * sig * u).astype(_BF)
        w2 = w2s_ref[0].astype(_BF)
        y = jax.lax.dot_general(h, w2, (((1,), (1,)), ((), ())),
                                preferred_element_type=jnp.float32)
        ys_ref[...] = y * wsl_ref[...]


def _moe_grouped(be, nb, xs, wslot2, ws, w2s):
    ws_h = ws.reshape(2 * E, IM, HIDDEN)

    def _bi(b, nb_r):
        return jnp.minimum(b, nb_r[0] - 1)

    grid_spec = pltpu.PrefetchScalarGridSpec(
        num_scalar_prefetch=2,
        grid=(NBMAX,),
        in_specs=[
            pl.BlockSpec((BLK, HIDDEN),
                         lambda b, be_r, nb_r: (_bi(b, nb_r), 0)),
            pl.BlockSpec((BLK, 1),
                         lambda b, be_r, nb_r: (_bi(b, nb_r), 0)),
            pl.BlockSpec((1, IM, HIDDEN),
                         lambda b, be_r, nb_r: (2 * be_r[_bi(b, nb_r)], 0, 0)),
            pl.BlockSpec((1, IM, HIDDEN),
                         lambda b, be_r, nb_r:
                         (2 * be_r[_bi(b, nb_r)] + 1, 0, 0)),
            pl.BlockSpec((1, HIDDEN, IM),
                         lambda b, be_r, nb_r: (be_r[_bi(b, nb_r)], 0, 0)),
        ],
        out_specs=pl.BlockSpec((BLK, HIDDEN), lambda b, be_r, nb_r: (b, 0)),
    )
    return pl.pallas_call(
        _moe_kernel,
        grid_spec=grid_spec,
        out_shape=jax.ShapeDtypeStruct((PADT, HIDDEN), jnp.float32),
    )(be, nb, xs, wslot2, ws_h, ws_h, w2s)


def _final_kernel(resid_ref, y1_ref, y2_ref, wpm_ref, out_ref):
    m = y1_ref[...] + y2_ref[...]
    out_ref[...] = resid_ref[...] + _rms(m, wpm_ref[...])


def _final(resid, y1, y2, w_post_moe):
    bt = 512
    return pl.pallas_call(
        _final_kernel,
        grid=(T // bt,),
        in_specs=[
            pl.BlockSpec((bt, HIDDEN), lambda t: (t, 0)),
            pl.BlockSpec((bt, HIDDEN), lambda t: (t, 0)),
            pl.BlockSpec((bt, HIDDEN), lambda t: (t, 0)),
            pl.BlockSpec((1, HIDDEN), lambda t: (0, 0)),
        ],
        out_specs=pl.BlockSpec((bt, HIDDEN), lambda t: (t, 0)),
        out_shape=jax.ShapeDtypeStruct((T, HIDDEN), jnp.float32),
    )(resid, y1, y2, w_post_moe)


def kernel(positions, hidden_states, wqkv, wo, gate_w, ws, w2s,
           w_pre_attn, w_post_attn, w_pre_moe, w_post_moe):
    posf = positions.astype(jnp.float32).reshape(T, 1)
    wpre = w_pre_attn.reshape(1, HIDDEN)
    wpost = w_post_attn.reshape(1, HIDDEN)
    wpremoe = w_pre_moe.reshape(1, HIDDEN)
    wpostmoe = w_post_moe.reshape(1, HIDDEN)

    trig = _rope_table(posf)
    qkv = _qkv_rope(trig, hidden_states, wpre, wqkv)
    o = _attention(qkv)
    resid, xm, logt = _post_attn(o, hidden_states, wo, wpost, wpremoe, gate_w)
    dispi, dispw = _dispatch_indices(logt)
    xs, wslot = _sc_dispatch(xm, dispi, dispw)
    be = dispi[2, :NBMAX]
    nb = dispi[3, :1]
    ys = _moe_grouped(be, nb, xs, wslot.reshape(PADT, 1), ws, w2s)
    y1, y2 = _sc_combine(ys, dispi)
    out = _final(resid, y1, y2, wpostmoe)
    return out, resid
